# Initial kernel scaffold; baseline (speedup 1.0000x reference)
import jax
import jax.numpy as jnp
from jax.experimental import pallas as pl


def kernel(x, edge_index, batch, W1, b1, W2, b2, W3, b3, W4, b4, cap_weight):
    raise NotImplementedError("write your pallas kernel here")


# speedup vs baseline: 8.5006x; 8.5006x over previous
import functools

import jax
import jax.numpy as jnp
import numpy as np
from jax import lax
from jax.experimental import pallas as pl
from jax.experimental.pallas import tpu as pltpu
from jax.experimental.pallas import tpu_sc as plsc

N = 10000
E = 320000
D_IN = 128
B = 128
K = 30
O = 10
L = 16
ILEN = 97
NITER = 3

NC = 2
NS = 16
NW = NC * NS

EPW = E // NW
ECH = 80
ET = EPW // ECH

NPAD = 10240
PW = NPAD // NW
NR = 10112
ZR = NR // NS
SLOTS = 4096
DUMP = B * K
PR = SLOTS // NW
HROWS = 10256

_f32 = jnp.float32
_i32 = jnp.int32


@functools.cache
def _mesh():
    return plsc.VectorSubcoreMesh(
        core_axis_name="c", subcore_axis_name="s",
        num_cores=NC, num_subcores=NS)


@functools.cache
def _build_deg():
    @functools.partial(
        pl.kernel,
        out_type=jax.ShapeDtypeStruct((NW, NPAD), _f32),
        mesh=_mesh(),
        compiler_params=pltpu.CompilerParams(needs_layout_passes=False, use_tc_tiling_on_sc=False),
        scratch_types=[
            pltpu.VMEM((EPW,), _i32),
            pltpu.VMEM((NPAD,), _f32),
        ],
    )
    def deg_kernel(dst_hbm, out_hbm, dstv, hist):
        cid = lax.axis_index("c")
        sid = lax.axis_index("s")
        wid = cid * NS + sid

        def zero(i, carry):
            hist[pl.ds(i * 16, 16)] = jnp.zeros((16,), _f32)
            return carry

        lax.fori_loop(0, NPAD // 16, zero, 0)
        pltpu.sync_copy(dst_hbm.at[pl.ds(wid * EPW, EPW)], dstv)
        ones = jnp.ones((16,), _f32)

        def body(i, carry):
            idx = dstv[pl.ds(i * 16, 16)]
            plsc.addupdate_scatter(hist, [idx], ones)
            return carry

        lax.fori_loop(0, EPW // 16, body, 0)
        pltpu.sync_copy(hist, out_hbm.at[wid])

    return deg_kernel


def _sc_deg(dst):
    return _build_deg()(dst)


@functools.cache
def _make_scatter(width):
    @functools.partial(
        pl.kernel,
        out_type=jax.ShapeDtypeStruct((NC, NR, width), _f32),
        mesh=_mesh(),
        compiler_params=pltpu.CompilerParams(needs_layout_passes=False, use_tc_tiling_on_sc=False),
        scratch_types=[
            pltpu.VMEM((ECH,), _i32),
            pltpu.VMEM((1, ECH), _i32),
            pltpu.VMEM((ECH, width), _f32),
            pltpu.VMEM((ZR, width), _f32),
            pltpu.VMEM_SHARED((NR, width), _f32),
            pltpu.SemaphoreType.DMA,
        ],
    )
    def scatter(g_hbm, src_hbm, dst_hbm, zeros_hbm, out_hbm,
                srcv, dstv, rows, zbuf, acc, sem):
        cid = lax.axis_index("c")
        sid = lax.axis_index("s")
        wid = cid * NS + sid
        pltpu.sync_copy(zeros_hbm, zbuf)
        pltpu.sync_copy(zbuf, acc.at[pl.ds(sid * ZR, ZR)])
        plsc.subcore_barrier()

        def body(t, carry):
            base = wid * EPW + t * ECH
            pltpu.sync_copy(src_hbm.at[pl.ds(base, ECH)], srcv)
            pltpu.async_copy(g_hbm.at[srcv], rows, sem).wait()
            pltpu.sync_copy(dst_hbm.at[pl.ds(base, ECH)], dstv.at[0])
            pltpu.sync_copy(rows, acc.at[dstv.at[0]], add=True)
            return carry

        lax.fori_loop(0, ET, body, 0)
        plsc.subcore_barrier()
        pltpu.sync_copy(acc.at[pl.ds(sid * ZR, ZR)], zbuf)
        pltpu.sync_copy(zbuf, out_hbm.at[cid, pl.ds(sid * ZR, ZR)])

    return scatter


def _sc_scatter32(g, src, dst, z):
    return _make_scatter(32)(g, src, dst, z)


def _sc_scatter16(g, src, dst, z):
    return _make_scatter(16)(g, src, dst, z)


@functools.cache
def _build_inv():
    @functools.partial(
        pl.kernel,
        out_type=jax.ShapeDtypeStruct((NW, SLOTS), _i32),
        mesh=_mesh(),
        compiler_params=pltpu.CompilerParams(needs_layout_passes=False, use_tc_tiling_on_sc=False),
        scratch_types=[
            pltpu.VMEM((PW,), _i32),
            pltpu.VMEM((SLOTS,), _i32),
        ],
    )
    def inv_kernel(slot_hbm, out_hbm, slotv, inv):
        cid = lax.axis_index("c")
        sid = lax.axis_index("s")
        wid = cid * NS + sid

        def zero(i, carry):
            inv[pl.ds(i * 16, 16)] = jnp.zeros((16,), _i32)
            return carry

        lax.fori_loop(0, SLOTS // 16, zero, 0)
        pltpu.sync_copy(slot_hbm.at[pl.ds(wid * PW, PW)], slotv)
        iota = lax.iota(_i32, 16)

        def body(t, carry):
            s16 = slotv[pl.ds(t * 16, 16)]
            ids = iota + (wid * PW + t * 16 + 1)
            plsc.store_scatter(inv, [s16], ids)
            return carry

        lax.fori_loop(0, PW // 16, body, 0)
        pltpu.sync_copy(inv, out_hbm.at[wid])

    return inv_kernel


def _sc_inv(slotp):
    return _build_inv()(slotp)


@functools.cache
def _build_pool():
    @functools.partial(
        pl.kernel,
        out_type=jax.ShapeDtypeStruct((SLOTS, D_IN), _f32),
        mesh=_mesh(),
        compiler_params=pltpu.CompilerParams(needs_layout_passes=False, use_tc_tiling_on_sc=False),
        scratch_types=[
            pltpu.VMEM((NW * PR,), _i32),
            pltpu.VMEM((PR,), _i32),
            pltpu.VMEM((PR, D_IN), _f32),
            pltpu.SemaphoreType.DMA,
        ],
    )
    def pool_kernel(parts_hbm, hpad_hbm, out_hbm, tmp, idxv, rows, sem):
        cid = lax.axis_index("c")
        sid = lax.axis_index("s")
        wid = cid * NS + sid
        base = wid * PR
        for p in range(NW):
            pltpu.sync_copy(parts_hbm.at[p, pl.ds(base, PR)],
                            tmp.at[pl.ds(p * PR, PR)])
        for j in range(PR // 16):
            v = tmp[pl.ds(j * 16, 16)]
            for p in range(1, NW):
                v = v + tmp[pl.ds(p * PR + j * 16, 16)]
            idxv[pl.ds(j * 16, 16)] = v
        pltpu.async_copy(hpad_hbm.at[idxv], rows, sem).wait()
        pltpu.sync_copy(rows, out_hbm.at[pl.ds(base, PR)])

    return pool_kernel


def _sc_pool(invparts, hpad):
    return _build_pool()(invparts, hpad)


def _tc_layer1(x, degparts, w1):
    blk = 400
    grid = N // blk

    def body(x_ref, dp_ref, w_ref, g_ref, dinv_ref):
        deg = jnp.sum(dp_ref[...], axis=1, keepdims=True) + 1.0
        dinv = 1.0 / jnp.sqrt(deg)
        g_ref[...] = jnp.dot(x_ref[...], w_ref[...],
                             preferred_element_type=_f32) * dinv
        dinv_ref[...] = dinv

    return pl.pallas_call(
        body,
        grid=(grid,),
        in_specs=[
            pl.BlockSpec((blk, D_IN), lambda i: (i, 0)),
            pl.BlockSpec((blk, NW), lambda i: (i, 0)),
            pl.BlockSpec((D_IN, 32), lambda i: (0, 0)),
        ],
        out_specs=[
            pl.BlockSpec((blk, 32), lambda i: (i, 0)),
            pl.BlockSpec((blk, 1), lambda i: (i, 0)),
        ],
        out_shape=[
            jax.ShapeDtypeStruct((N, 32), _f32),
            jax.ShapeDtypeStruct((N, 1), _f32),
        ],
    )(x, degparts, w1)


def _tc_layer(parts, g, dinv, b, w_next):
    blk = 400
    grid = N // blk
    win = g.shape[1]
    wout = w_next.shape[1]

    def body(p_ref, g_ref, d_ref, b_ref, w_ref, x_ref, gn_ref):
        p = p_ref[...]
        acc = p[0] + p[1] + g_ref[...]
        d = d_ref[...]
        xv = jnp.tanh(acc * d + b_ref[...])
        x_ref[...] = xv
        gn_ref[...] = jnp.dot(xv, w_ref[...], preferred_element_type=_f32) * d

    return pl.pallas_call(
        body,
        grid=(grid,),
        in_specs=[
            pl.BlockSpec((NC, blk, win), lambda i: (0, i, 0)),
            pl.BlockSpec((blk, win), lambda i: (i, 0)),
            pl.BlockSpec((blk, 1), lambda i: (i, 0)),
            pl.BlockSpec((1, win), lambda i: (0, 0)),
            pl.BlockSpec((win, wout), lambda i: (0, 0)),
        ],
        out_specs=[
            pl.BlockSpec((blk, win), lambda i: (i, 0)),
            pl.BlockSpec((blk, wout), lambda i: (i, 0)),
        ],
        out_shape=[
            jax.ShapeDtypeStruct((N, win), _f32),
            jax.ShapeDtypeStruct((N, wout), _f32),
        ],
    )(parts, g, dinv, b, w_next)


def _tc_last(parts, g, dinv, b):
    blk = 400
    grid = N // blk
    win = g.shape[1]

    def body(p_ref, g_ref, d_ref, b_ref, x_ref):
        p = p_ref[...]
        acc = p[0] + p[1] + g_ref[...]
        x_ref[...] = jnp.tanh(acc * d_ref[...] + b_ref[...])

    return pl.pallas_call(
        body,
        grid=(grid,),
        in_specs=[
            pl.BlockSpec((NC, blk, win), lambda i: (0, i, 0)),
            pl.BlockSpec((blk, win), lambda i: (i, 0)),
            pl.BlockSpec((blk, 1), lambda i: (i, 0)),
            pl.BlockSpec((1, win), lambda i: (0, 0)),
        ],
        out_specs=pl.BlockSpec((blk, win), lambda i: (i, 0)),
        out_shape=jax.ShapeDtypeStruct((N, win), _f32),
    )(parts, g, dinv, b)


def _tc_rank(batch_col, v_col, batch_row, v_row):
    blk = 400
    grid = N // blk
    jblk = 500

    def body(bc_ref, vc_ref, br_ref, vr_ref, slot_ref):
        bc = bc_ref[...]
        kci = bc.astype(_f32) * 4.0 - vc_ref[...]
        br = br_ref[...]
        kr = br.astype(_f32) * 4.0 - vr_ref[...]
        i0 = pl.program_id(0) * blk
        irow = i0 + lax.broadcasted_iota(_i32, (blk, 1), 0)
        cnt = jnp.zeros((blk, 1), _f32)
        starts = jnp.zeros((blk, 1), _f32)
        for jb in range(N // jblk):
            krb = kr[:, jb * jblk:(jb + 1) * jblk]
            brb = br[:, jb * jblk:(jb + 1) * jblk]
            jrow = jb * jblk + lax.broadcasted_iota(_i32, (1, jblk), 1)
            lt = krb < kci
            tie = (krb == kci) & (jrow < irow)
            cnt = cnt + jnp.sum((lt | tie).astype(_f32), axis=1,
                                keepdims=True)
            starts = starts + jnp.sum((brb < bc).astype(_f32), axis=1,
                                      keepdims=True)
        rank = (cnt - starts).astype(_i32)
        slot_ref[...] = jnp.where(rank < K, bc * K + rank, DUMP)

    return pl.pallas_call(
        body,
        grid=(grid,),
        in_specs=[
            pl.BlockSpec((blk, 1), lambda i: (i, 0)),
            pl.BlockSpec((blk, 1), lambda i: (i, 0)),
            pl.BlockSpec((1, N), lambda i: (0, 0)),
            pl.BlockSpec((1, N), lambda i: (0, 0)),
        ],
        out_specs=pl.BlockSpec((blk, 1), lambda i: (i, 0)),
        out_shape=jax.ShapeDtypeStruct((N, 1), _i32),
    )(batch_col, v_col, batch_row, v_row)


_G = np.kron(np.eye(O, dtype=np.float32),
             np.ones((L, 1), np.float32))
_GT = np.kron(np.eye(O, dtype=np.float32),
              np.ones((1, L), np.float32))
_SEL = np.kron(np.eye(B, dtype=np.float32),
               np.ones((1, K), np.float32))
_SELT = np.kron(np.eye(B, dtype=np.float32),
                np.ones((K, 1), np.float32))


def _tc_caps(pooled, m):
    def body(p_ref, m_ref, g_ref, gt_ref, sel_ref, selt_ref, cls_ref):
        mm = lambda a, b: jnp.dot(a, b, preferred_element_type=_f32)
        g = g_ref[...]
        gt = gt_ref[...]
        sel = sel_ref[...]
        selt = selt_ref[...]
        pr = mm(p_ref[...], m_ref[...])
        pp = mm(pr * pr, g)
        out = mm(sel, pr) / float(K)
        for _ in range(NITER):
            outr = mm(selt, out)
            dot = mm(pr * outr, g)
            oo = mm(out * out, g)
            oor = mm(selt, oo)
            sim = dot / (pp + oor - dot)
            e = jnp.exp(sim)
            denom = mm(selt, mm(sel, e))
            probs = e / denom
            out = mm(sel, pr * mm(probs, gt))
        n2 = mm(out * out, g)
        scale = (n2 / (1.0 + n2)) / jnp.sqrt(n2 + 1e-12)
        out = out * mm(scale, gt)
        cls_ref[...] = jnp.sqrt(mm(out * out, g))

    return pl.pallas_call(
        body,
        out_shape=jax.ShapeDtypeStruct((B, O), _f32),
    )(pooled, m, jnp.asarray(_G), jnp.asarray(_GT),
      jnp.asarray(_SEL), jnp.asarray(_SELT))


@jax.jit
def kernel(x, edge_index, batch, W1, b1, W2, b2, W3, b3, W4, b4, cap_weight):
    src = edge_index[0]
    dst = edge_index[1]
    z32 = jnp.zeros((ZR, 32), _f32)
    z16 = jnp.zeros((ZR, 16), _f32)

    degparts = _sc_deg(dst)
    g1, dinv = _tc_layer1(x, degparts[:, :N].T, W1)
    p1 = _sc_scatter32(g1, src, dst, z32)
    x1, g2 = _tc_layer(p1[:, :N], g1, dinv, b1.reshape(1, 32), W2)
    p2 = _sc_scatter32(g2, src, dst, z32)
    x2, g3 = _tc_layer(p2[:, :N], g2, dinv, b2.reshape(1, 32), W3)
    p3 = _sc_scatter32(g3, src, dst, z32)
    w4p = jnp.pad(W4, ((0, 0), (0, 15)))
    x3, g4 = _tc_layer(p3[:, :N], g3, dinv, b3.reshape(1, 32), w4p)
    p4 = _sc_scatter16(g4, src, dst, z16)
    b4p = jnp.pad(b4, (0, 15)).reshape(1, 16)
    x4 = _tc_last(p4[:, :N], g4, dinv, b4p)
    v = x4[:, :1]

    slot = _tc_rank(batch[:, None], v, batch[None, :], v.reshape(1, N))
    slotp = jnp.pad(slot[:, 0], (0, NPAD - N), constant_values=DUMP)
    invparts = _sc_inv(slotp)

    h = jnp.concatenate([x1, x2, x3, v], axis=1)
    hpad = jnp.pad(h, ((1, HROWS - 1 - N), (0, D_IN - ILEN)))
    pooled = _sc_pool(invparts, hpad)

    m = jnp.pad(cap_weight.transpose(2, 0, 1).reshape(ILEN, O * L),
                ((0, D_IN - ILEN), (0, 0)))
    return _tc_caps(pooled[:DUMP], m)


# speedup vs baseline: 21.3328x; 2.5096x over previous
import functools

import jax
import jax.numpy as jnp
import numpy as np
from jax import lax
from jax.experimental import pallas as pl
from jax.experimental.pallas import tpu as pltpu
from jax.experimental.pallas import tpu_sc as plsc

N = 10000
E = 320000
D_IN = 128
B = 128
K = 30
O = 10
L = 16
ILEN = 97
NITER = 3

NC = 2
NS = 16
NW = NC * NS

EPW = E // NW
ECH = 80
ET = EPW // ECH

NPAD = 10240
PW = NPAD // NW
NR = 10112
ZR = NR // NS
SLOTS = 4096
DUMP = B * K
PR = SLOTS // NW
HROWS = 10256

_f32 = jnp.float32
_i32 = jnp.int32


@functools.cache
def _mesh():
    return plsc.VectorSubcoreMesh(
        core_axis_name="c", subcore_axis_name="s",
        num_cores=NC, num_subcores=NS)


@functools.cache
def _build_deg():
    @functools.partial(
        pl.kernel,
        out_type=jax.ShapeDtypeStruct((NW, NPAD), _f32),
        mesh=_mesh(),
        compiler_params=pltpu.CompilerParams(needs_layout_passes=False, use_tc_tiling_on_sc=False),
        scratch_types=[
            pltpu.VMEM((EPW,), _i32),
            pltpu.VMEM((NPAD,), _f32),
        ],
    )
    def deg_kernel(dst_hbm, out_hbm, dstv, hist):
        cid = lax.axis_index("c")
        sid = lax.axis_index("s")
        wid = cid * NS + sid

        def zero(i, carry):
            hist[pl.ds(i * 16, 16)] = jnp.zeros((16,), _f32)
            return carry

        lax.fori_loop(0, NPAD // 16, zero, 0)
        pltpu.sync_copy(dst_hbm.at[pl.ds(wid * EPW, EPW)], dstv)
        ones = jnp.ones((16,), _f32)

        def body(i, carry):
            idx = dstv[pl.ds(i * 16, 16)]
            plsc.addupdate_scatter(hist, [idx], ones)
            return carry

        lax.fori_loop(0, EPW // 16, body, 0)
        pltpu.sync_copy(hist, out_hbm.at[wid])

    return deg_kernel


def _sc_deg(dst):
    return _build_deg()(dst)


@functools.cache
def _make_scatter(width):
    @functools.partial(
        pl.kernel,
        out_type=jax.ShapeDtypeStruct((NC, NR, width), _f32),
        mesh=_mesh(),
        compiler_params=pltpu.CompilerParams(needs_layout_passes=False, use_tc_tiling_on_sc=False),
        scratch_types=[
            pltpu.VMEM((ET, ECH), _i32),
            pltpu.VMEM((ET, ECH), _i32),
            pltpu.VMEM((2, ECH, width), _f32),
            pltpu.VMEM((ZR, width), _f32),
            pltpu.VMEM_SHARED((NR, width), _f32),
            pltpu.SemaphoreType.DMA,
            pltpu.SemaphoreType.DMA,
        ],
    )
    def scatter(g_hbm, src_hbm, dst_hbm, zeros_hbm, out_hbm,
                srcall, dstall, rows, zbuf, acc, sem0, sem1):
        cid = lax.axis_index("c")
        sid = lax.axis_index("s")
        wid = cid * NS + sid
        sems = (sem0, sem1)
        pltpu.sync_copy(src_hbm.at[pl.ds(wid * ET, ET)], srcall)
        pltpu.sync_copy(dst_hbm.at[pl.ds(wid * ET, ET)], dstall)
        pltpu.sync_copy(zeros_hbm, zbuf)
        pltpu.sync_copy(zbuf, acc.at[pl.ds(sid * ZR, ZR)])
        plsc.subcore_barrier()

        def start_g(t, b):
            pltpu.async_copy(g_hbm.at[srcall.at[t]], rows.at[b], sems[b])

        def wait_g(t, b):
            pltpu.make_async_copy(
                g_hbm.at[srcall.at[t]], rows.at[b], sems[b]).wait()

        start_g(0, 0)
        start_g(1, 1)

        def pair(u, carry):
            t0 = 2 * u
            wait_g(t0, 0)
            pltpu.sync_copy(rows.at[0], acc.at[dstall.at[t0]], add=True)

            @pl.when(t0 + 2 < ET)
            def _():
                start_g(t0 + 2, 0)

            @pl.when(t0 + 1 < ET)
            def _():
                wait_g(t0 + 1, 1)
                pltpu.sync_copy(rows.at[1], acc.at[dstall.at[t0 + 1]],
                                add=True)

                @pl.when(t0 + 3 < ET)
                def _():
                    start_g(t0 + 3, 1)

            return carry

        lax.fori_loop(0, (ET + 1) // 2, pair, 0)
        plsc.subcore_barrier()
        pltpu.sync_copy(acc.at[pl.ds(sid * ZR, ZR)], zbuf)
        pltpu.sync_copy(zbuf, out_hbm.at[cid, pl.ds(sid * ZR, ZR)])

    return scatter


def _sc_scatter32(g, src, dst, z):
    return _make_scatter(32)(g, src.reshape(NW * ET, ECH),
                             dst.reshape(NW * ET, ECH), z)


def _sc_scatter16(g, src, dst, z):
    return _make_scatter(16)(g, src.reshape(NW * ET, ECH),
                             dst.reshape(NW * ET, ECH), z)


@functools.cache
def _build_inv():
    @functools.partial(
        pl.kernel,
        out_type=jax.ShapeDtypeStruct((NW, SLOTS), _i32),
        mesh=_mesh(),
        compiler_params=pltpu.CompilerParams(needs_layout_passes=False, use_tc_tiling_on_sc=False),
        scratch_types=[
            pltpu.VMEM((PW,), _i32),
            pltpu.VMEM((SLOTS,), _i32),
        ],
    )
    def inv_kernel(slot_hbm, out_hbm, slotv, inv):
        cid = lax.axis_index("c")
        sid = lax.axis_index("s")
        wid = cid * NS + sid

        def zero(i, carry):
            inv[pl.ds(i * 16, 16)] = jnp.zeros((16,), _i32)
            return carry

        lax.fori_loop(0, SLOTS // 16, zero, 0)
        pltpu.sync_copy(slot_hbm.at[pl.ds(wid * PW, PW)], slotv)
        iota = lax.iota(_i32, 16)

        def body(t, carry):
            s16 = slotv[pl.ds(t * 16, 16)]
            ids = iota + (wid * PW + t * 16 + 1)
            plsc.store_scatter(inv, [s16], ids)
            return carry

        lax.fori_loop(0, PW // 16, body, 0)
        pltpu.sync_copy(inv, out_hbm.at[wid])

    return inv_kernel


def _sc_inv(slotp):
    return _build_inv()(slotp)


@functools.cache
def _build_pool():
    @functools.partial(
        pl.kernel,
        out_type=jax.ShapeDtypeStruct((SLOTS, D_IN), _f32),
        mesh=_mesh(),
        compiler_params=pltpu.CompilerParams(needs_layout_passes=False, use_tc_tiling_on_sc=False),
        scratch_types=[
            pltpu.VMEM((NW * PR,), _i32),
            pltpu.VMEM((PR,), _i32),
            pltpu.VMEM((PR, D_IN), _f32),
            pltpu.SemaphoreType.DMA,
        ],
    )
    def pool_kernel(parts_hbm, hpad_hbm, out_hbm, tmp, idxv, rows, sem):
        cid = lax.axis_index("c")
        sid = lax.axis_index("s")
        wid = cid * NS + sid
        base = wid * PR
        for p in range(NW):
            pltpu.sync_copy(parts_hbm.at[p, pl.ds(base, PR)],
                            tmp.at[pl.ds(p * PR, PR)])
        for j in range(PR // 16):
            v = tmp[pl.ds(j * 16, 16)]
            for p in range(1, NW):
                v = v + tmp[pl.ds(p * PR + j * 16, 16)]
            idxv[pl.ds(j * 16, 16)] = v
        pltpu.async_copy(hpad_hbm.at[idxv], rows, sem).wait()
        pltpu.sync_copy(rows, out_hbm.at[pl.ds(base, PR)])

    return pool_kernel


def _sc_pool(invparts, hpad):
    return _build_pool()(invparts, hpad)


def _tc_layer1(x, degparts, w1):
    blk = 400
    grid = N // blk

    def body(x_ref, dp_ref, w_ref, g_ref, dinv_ref):
        deg = jnp.sum(dp_ref[...], axis=1, keepdims=True) + 1.0
        dinv = 1.0 / jnp.sqrt(deg)
        g_ref[...] = jnp.dot(x_ref[...], w_ref[...],
                             preferred_element_type=_f32) * dinv
        dinv_ref[...] = dinv

    return pl.pallas_call(
        body,
        grid=(grid,),
        in_specs=[
            pl.BlockSpec((blk, D_IN), lambda i: (i, 0)),
            pl.BlockSpec((blk, NW), lambda i: (i, 0)),
            pl.BlockSpec((D_IN, 32), lambda i: (0, 0)),
        ],
        out_specs=[
            pl.BlockSpec((blk, 32), lambda i: (i, 0)),
            pl.BlockSpec((blk, 1), lambda i: (i, 0)),
        ],
        out_shape=[
            jax.ShapeDtypeStruct((N, 32), _f32),
            jax.ShapeDtypeStruct((N, 1), _f32),
        ],
    )(x, degparts, w1)


def _tc_layer(parts, g, dinv, b, w_next):
    blk = 400
    grid = N // blk
    win = g.shape[1]
    wout = w_next.shape[1]

    def body(p_ref, g_ref, d_ref, b_ref, w_ref, x_ref, gn_ref):
        p = p_ref[...]
        acc = p[0] + p[1] + g_ref[...]
        d = d_ref[...]
        xv = jnp.tanh(acc * d + b_ref[...])
        x_ref[...] = xv
        gn_ref[...] = jnp.dot(xv, w_ref[...], preferred_element_type=_f32) * d

    return pl.pallas_call(
        body,
        grid=(grid,),
        in_specs=[
            pl.BlockSpec((NC, blk, win), lambda i: (0, i, 0)),
            pl.BlockSpec((blk, win), lambda i: (i, 0)),
            pl.BlockSpec((blk, 1), lambda i: (i, 0)),
            pl.BlockSpec((1, win), lambda i: (0, 0)),
            pl.BlockSpec((win, wout), lambda i: (0, 0)),
        ],
        out_specs=[
            pl.BlockSpec((blk, win), lambda i: (i, 0)),
            pl.BlockSpec((blk, wout), lambda i: (i, 0)),
        ],
        out_shape=[
            jax.ShapeDtypeStruct((N, win), _f32),
            jax.ShapeDtypeStruct((N, wout), _f32),
        ],
    )(parts, g, dinv, b, w_next)


def _tc_last(parts, g, dinv, b):
    blk = 400
    grid = N // blk
    win = g.shape[1]

    def body(p_ref, g_ref, d_ref, b_ref, x_ref):
        p = p_ref[...]
        acc = p[0] + p[1] + g_ref[...]
        x_ref[...] = jnp.tanh(acc * d_ref[...] + b_ref[...])

    return pl.pallas_call(
        body,
        grid=(grid,),
        in_specs=[
            pl.BlockSpec((NC, blk, win), lambda i: (0, i, 0)),
            pl.BlockSpec((blk, win), lambda i: (i, 0)),
            pl.BlockSpec((blk, 1), lambda i: (i, 0)),
            pl.BlockSpec((1, win), lambda i: (0, 0)),
        ],
        out_specs=pl.BlockSpec((blk, win), lambda i: (i, 0)),
        out_shape=jax.ShapeDtypeStruct((N, win), _f32),
    )(parts, g, dinv, b)


def _tc_rank(batch_col, v_col, batch_row, v_row):
    blk = 400
    grid = N // blk
    jblk = 500

    def body(bc_ref, vc_ref, br_ref, vr_ref, slot_ref, cnt_ref, st_ref):
        bc = bc_ref[...]
        kci = bc.astype(_f32) * 4.0 - vc_ref[...]
        br = br_ref[...]
        kr = br.astype(_f32) * 4.0 - vr_ref[...]
        i0 = pl.program_id(0) * blk
        irow = i0 + lax.broadcasted_iota(_i32, (blk, 1), 0)
        blo = jnp.min(bc)
        bhi = jnp.max(bc)
        c0 = jnp.sum((br < blo).astype(_i32)) // jblk
        c1 = (jnp.sum((br <= bhi).astype(_i32)) + jblk - 1) // jblk
        cnt_ref[...] = jnp.zeros((blk, 1), _f32)
        st_ref[...] = jnp.zeros((blk, 1), _f32)
        for jb in range(NPAD // jblk):
            @pl.when((jb >= c0) & (jb < c1))
            def _():
                krb = kr[:, jb * jblk:(jb + 1) * jblk]
                brb = br[:, jb * jblk:(jb + 1) * jblk]
                jrow = jb * jblk + lax.broadcasted_iota(_i32, (1, jblk), 1)
                lt = krb < kci
                tie = (krb == kci) & (jrow < irow)
                cnt_ref[...] += jnp.sum((lt | tie).astype(_f32), axis=1,
                                        keepdims=True)
                st_ref[...] += jnp.sum((brb < bc).astype(_f32), axis=1,
                                       keepdims=True)
        rank = (cnt_ref[...] - st_ref[...]).astype(_i32)
        slot_ref[...] = jnp.where(rank < K, bc * K + rank, DUMP)

    return pl.pallas_call(
        body,
        grid=(grid,),
        in_specs=[
            pl.BlockSpec((blk, 1), lambda i: (i, 0)),
            pl.BlockSpec((blk, 1), lambda i: (i, 0)),
            pl.BlockSpec((1, NPAD), lambda i: (0, 0)),
            pl.BlockSpec((1, NPAD), lambda i: (0, 0)),
        ],
        out_specs=pl.BlockSpec((blk, 1), lambda i: (i, 0)),
        out_shape=jax.ShapeDtypeStruct((N, 1), _i32),
        scratch_shapes=[
            pltpu.VMEM((blk, 1), _f32),
            pltpu.VMEM((blk, 1), _f32),
        ],
    )(batch_col, v_col, batch_row, v_row)


_G = np.kron(np.eye(O, dtype=np.float32),
             np.ones((L, 1), np.float32))
_GT = np.kron(np.eye(O, dtype=np.float32),
              np.ones((1, L), np.float32))
_SEL = np.kron(np.eye(B, dtype=np.float32),
               np.ones((1, K), np.float32))
_SELT = np.kron(np.eye(B, dtype=np.float32),
                np.ones((K, 1), np.float32))


def _tc_caps(pooled, m):
    def body(p_ref, m_ref, g_ref, gt_ref, sel_ref, selt_ref, cls_ref):
        mm = lambda a, b: jnp.dot(a, b, preferred_element_type=_f32)
        g = g_ref[...]
        gt = gt_ref[...]
        sel = sel_ref[...]
        selt = selt_ref[...]
        pr = mm(p_ref[...], m_ref[...])
        pp = mm(pr * pr, g)
        out = mm(sel, pr) / float(K)
        for _ in range(NITER):
            outr = mm(selt, out)
            dot = mm(pr * outr, g)
            oo = mm(out * out, g)
            oor = mm(selt, oo)
            sim = dot / (pp + oor - dot)
            e = jnp.exp(sim)
            denom = mm(selt, mm(sel, e))
            probs = e / denom
            out = mm(sel, pr * mm(probs, gt))
        n2 = mm(out * out, g)
        scale = (n2 / (1.0 + n2)) / jnp.sqrt(n2 + 1e-12)
        out = out * mm(scale, gt)
        cls_ref[...] = jnp.sqrt(mm(out * out, g))

    return pl.pallas_call(
        body,
        out_shape=jax.ShapeDtypeStruct((B, O), _f32),
    )(pooled, m, jnp.asarray(_G), jnp.asarray(_GT),
      jnp.asarray(_SEL), jnp.asarray(_SELT))


@jax.jit
def kernel(x, edge_index, batch, W1, b1, W2, b2, W3, b3, W4, b4, cap_weight):
    src = edge_index[0]
    dst = edge_index[1]
    z32 = jnp.zeros((ZR, 32), _f32)
    z16 = jnp.zeros((ZR, 16), _f32)

    degparts = _sc_deg(dst)
    g1, dinv = _tc_layer1(x, degparts[:, :N].T, W1)
    p1 = _sc_scatter32(g1, src, dst, z32)
    x1, g2 = _tc_layer(p1[:, :N], g1, dinv, b1.reshape(1, 32), W2)
    p2 = _sc_scatter32(g2, src, dst, z32)
    x2, g3 = _tc_layer(p2[:, :N], g2, dinv, b2.reshape(1, 32), W3)
    p3 = _sc_scatter32(g3, src, dst, z32)
    w4p = jnp.pad(W4, ((0, 0), (0, 15)))
    x3, g4 = _tc_layer(p3[:, :N], g3, dinv, b3.reshape(1, 32), w4p)
    p4 = _sc_scatter16(g4, src, dst, z16)
    b4p = jnp.pad(b4, (0, 15)).reshape(1, 16)
    x4 = _tc_last(p4[:, :N], g4, dinv, b4p)
    v = x4[:, :1]

    brow = jnp.pad(batch[None, :], ((0, 0), (0, NPAD - N)),
                   constant_values=1 << 20)
    vrow = jnp.pad(v.reshape(1, N), ((0, 0), (0, NPAD - N)))
    slot = _tc_rank(batch[:, None], v, brow, vrow)
    slotp = jnp.pad(slot[:, 0], (0, NPAD - N), constant_values=DUMP)
    invparts = _sc_inv(slotp)

    h = jnp.concatenate([x1, x2, x3, v], axis=1)
    hpad = jnp.pad(h, ((1, HROWS - 1 - N), (0, D_IN - ILEN)))
    pooled = _sc_pool(invparts, hpad)

    m = jnp.pad(cap_weight.transpose(2, 0, 1).reshape(ILEN, O * L),
                ((0, D_IN - ILEN), (0, 0)))
    return _tc_caps(pooled[:DUMP], m)
